# Initial kernel scaffold; baseline (speedup 1.0000x reference)
import jax
import jax.numpy as jnp
from jax.experimental import pallas as pl


def kernel(P, Ps, sample_prob):
    raise NotImplementedError("write your pallas kernel here")


# speedup vs baseline: 2.9993x; 2.9993x over previous
import functools

import jax
import jax.numpy as jnp
from jax import lax
from jax.experimental import pallas as pl
from jax.experimental.pallas import tpu as pltpu
from jax.experimental.pallas import tpu_sc as plsc

_N_P = 16384
_N_S = 4096
_BR = 512
_GRID = _N_P // _BR

_NC = 2
_NS = 16
_NW = _NC * _NS
_B_PER_W = _N_P // _NW
_LANES = 16
_N_VEC = _B_PER_W // _LANES


def _tc_body(p_ref, pst_ref, sp_ref, rowdist_ref, rowarg_ref, colterm_ref,
             colmin_ref):
    i = pl.program_id(0)
    p = p_ref[...]
    pst = pst_ref[...]
    pn = jnp.sum(p * p, axis=1, keepdims=True)
    psn = jnp.sum(pst * pst, axis=0, keepdims=True)
    m = jnp.dot(p, pst, preferred_element_type=jnp.float32)
    sq = (pn + psn) - 2.0 * m

    rowmin = jnp.min(sq, axis=1, keepdims=True)
    iota = lax.broadcasted_iota(jnp.int32, sq.shape, 1)
    masked = jnp.where(sq == rowmin, iota, _N_S)
    rowarg = jnp.min(masked, axis=1, keepdims=True)
    rowdist_ref[...] = jnp.sqrt(jnp.maximum(rowmin, 0.0))
    rowarg_ref[...] = rowarg

    bmin = jnp.min(sq, axis=0, keepdims=True)

    @pl.when(i == 0)
    def _():
        colmin_ref[...] = bmin

    @pl.when(i > 0)
    def _():
        colmin_ref[...] = jnp.minimum(colmin_ref[...], bmin)

    @pl.when(i == _GRID - 1)
    def _():
        cold = jnp.sqrt(jnp.maximum(colmin_ref[...], 0.0))
        colterm_ref[...] = jnp.sum(cold * sp_ref[...]).reshape(1, 1)


_tc_call = pl.pallas_call(
    _tc_body,
    grid=(_GRID,),
    in_specs=[
        pl.BlockSpec((_BR, 3), lambda i: (i, 0)),
        pl.BlockSpec((3, _N_S), lambda i: (0, 0)),
        pl.BlockSpec((1, _N_S), lambda i: (0, 0)),
    ],
    out_specs=[
        pl.BlockSpec((_BR, 1), lambda i: (i, 0)),
        pl.BlockSpec((_BR, 1), lambda i: (i, 0)),
        pl.BlockSpec((1, 1), lambda i: (0, 0)),
    ],
    out_shape=[
        jax.ShapeDtypeStruct((_N_P, 1), jnp.float32),
        jax.ShapeDtypeStruct((_N_P, 1), jnp.int32),
        jax.ShapeDtypeStruct((1, 1), jnp.float32),
    ],
    scratch_shapes=[pltpu.VMEM((1, _N_S), jnp.float32)],
    compiler_params=pltpu.CompilerParams(
        dimension_semantics=("arbitrary",)),
)


@functools.cache
def _make_sc_gather():
    @functools.partial(
        pl.kernel,
        out_type=jax.ShapeDtypeStruct((_NW, _LANES), jnp.float32),
        mesh=plsc.VectorSubcoreMesh(core_axis_name="c", subcore_axis_name="s",
                                    num_cores=_NC, num_subcores=_NS),
        scratch_types=[
            pltpu.VMEM((_B_PER_W,), jnp.int32),
            pltpu.VMEM((_B_PER_W,), jnp.float32),
            pltpu.VMEM((_N_S,), jnp.float32),
            pltpu.VMEM((_LANES,), jnp.float32),
        ],
        compiler_params=pltpu.CompilerParams(needs_layout_passes=False),
    )
    def _sc_gather(dist_hbm, idx_hbm, sp_hbm, out_hbm, idx_v, dist_v, sp_v,
                   acc_v):
        wid = lax.axis_index("s") * _NC + lax.axis_index("c")
        base = wid * _B_PER_W
        pltpu.sync_copy(idx_hbm.at[pl.ds(base, _B_PER_W)], idx_v)
        pltpu.sync_copy(dist_hbm.at[pl.ds(base, _B_PER_W)], dist_v)
        pltpu.sync_copy(sp_hbm, sp_v)

        def body(i, acc):
            idx = idx_v[pl.ds(i * _LANES, _LANES)]
            d = dist_v[pl.ds(i * _LANES, _LANES)]
            vals = plsc.load_gather(sp_v, [idx])
            return acc + d * vals

        acc = lax.fori_loop(0, _N_VEC, body,
                            jnp.zeros((_LANES,), jnp.float32))
        acc_v[...] = acc
        pltpu.sync_copy(acc_v, out_hbm.at[wid])

    return _sc_gather


def kernel(P, Ps, sample_prob):
    pst = Ps.T
    sp2 = sample_prob.reshape(1, _N_S)
    rowdist, rowarg, colterm = _tc_call(P, pst, sp2)
    partials = _make_sc_gather()(rowdist.reshape(_N_P), rowarg.reshape(_N_P),
                                 sample_prob)
    return colterm[0, 0] + jnp.sum(partials)


# speedup vs baseline: 3.2493x; 1.0833x over previous
import functools

import jax
import jax.numpy as jnp
from jax import lax
from jax.experimental import pallas as pl
from jax.experimental.pallas import tpu as pltpu
from jax.experimental.pallas import tpu_sc as plsc

_N_P = 16384
_N_S = 4096
_BR = 256
_GRID = _N_P // _BR

_NC = 2
_NS = 16
_NW = _NC * _NS
_B_PER_W = _N_P // _NW
_LANES = 16
_N_VEC = _B_PER_W // _LANES


def _tc_body(p_ref, pst_ref, sp_ref, rowdist_ref, rowarg_ref, colterm_ref,
             colmin_ref):
    i = pl.program_id(0)
    p = p_ref[...]
    p2 = p * -2.0
    pst = pst_ref[...]
    pn = jnp.sum(p * p, axis=1)
    psn = jnp.sum(pst * pst, axis=0, keepdims=True)
    m2 = jnp.dot(p2, pst, preferred_element_type=jnp.float32)

    t = psn + m2
    rowmin = jnp.min(t, axis=1) + pn
    rowarg = jnp.argmin(t, axis=1)
    rowdist_ref[...] = jnp.sqrt(jnp.maximum(rowmin, 0.0))
    rowarg_ref[...] = rowarg

    u = pn[:, None] + m2
    bmin = jnp.min(u, axis=0, keepdims=True)

    @pl.when(i == 0)
    def _():
        colmin_ref[...] = bmin

    @pl.when(i > 0)
    def _():
        colmin_ref[...] = jnp.minimum(colmin_ref[...], bmin)

    @pl.when(i == _GRID - 1)
    def _():
        cold = jnp.sqrt(jnp.maximum(colmin_ref[...] + psn, 0.0))
        colterm_ref[...] = jnp.sum(cold * sp_ref[...]).reshape(1, 1)


_tc_call = pl.pallas_call(
    _tc_body,
    grid=(_GRID,),
    in_specs=[
        pl.BlockSpec((_BR, 3), lambda i: (i, 0)),
        pl.BlockSpec((3, _N_S), lambda i: (0, 0)),
        pl.BlockSpec((1, _N_S), lambda i: (0, 0)),
    ],
    out_specs=[
        pl.BlockSpec((_BR,), lambda i: (i,)),
        pl.BlockSpec((_BR,), lambda i: (i,)),
        pl.BlockSpec((1, 1), lambda i: (0, 0)),
    ],
    out_shape=[
        jax.ShapeDtypeStruct((_N_P,), jnp.float32),
        jax.ShapeDtypeStruct((_N_P,), jnp.int32),
        jax.ShapeDtypeStruct((1, 1), jnp.float32),
    ],
    scratch_shapes=[
        pltpu.VMEM((1, _N_S), jnp.float32),
    ],
    compiler_params=pltpu.CompilerParams(
        dimension_semantics=("arbitrary",)),
)


@functools.cache
def _make_sc_gather():
    @functools.partial(
        pl.kernel,
        out_type=jax.ShapeDtypeStruct((_NW, _LANES), jnp.float32),
        mesh=plsc.VectorSubcoreMesh(core_axis_name="c", subcore_axis_name="s",
                                    num_cores=_NC, num_subcores=_NS),
        scratch_types=[
            pltpu.VMEM((_B_PER_W,), jnp.int32),
            pltpu.VMEM((_B_PER_W,), jnp.float32),
            pltpu.VMEM((_N_S,), jnp.float32),
            pltpu.VMEM((_LANES,), jnp.float32),
            pltpu.SemaphoreType.DMA,
            pltpu.SemaphoreType.DMA,
            pltpu.SemaphoreType.DMA,
        ],
        compiler_params=pltpu.CompilerParams(needs_layout_passes=False),
    )
    def _sc_gather(dist_hbm, idx_hbm, sp_hbm, out_hbm,
                   idx_v, dist_v, sp_v, acc_v, s1, s2, s3):
        wid = lax.axis_index("s") * _NC + lax.axis_index("c")
        base = wid * _B_PER_W
        c1 = pltpu.make_async_copy(idx_hbm.at[pl.ds(base, _B_PER_W)],
                                   idx_v, s1)
        c2 = pltpu.make_async_copy(dist_hbm.at[pl.ds(base, _B_PER_W)],
                                   dist_v, s2)
        c3 = pltpu.make_async_copy(sp_hbm, sp_v, s3)
        c1.start()
        c2.start()
        c3.start()
        c1.wait()
        c2.wait()
        c3.wait()

        def body(i, acc):
            idx = idx_v[pl.ds(i * _LANES, _LANES)]
            d = dist_v[pl.ds(i * _LANES, _LANES)]
            vals = plsc.load_gather(sp_v, [idx])
            return acc + d * vals

        acc = lax.fori_loop(0, _N_VEC, body,
                            jnp.zeros((_LANES,), jnp.float32))
        acc_v[...] = acc
        pltpu.sync_copy(acc_v, out_hbm.at[wid])

    return _sc_gather


def kernel(P, Ps, sample_prob):
    sp2 = sample_prob.reshape(1, _N_S)
    rowdist, rowarg, colterm = _tc_call(P, Ps.T, sp2)
    partials = _make_sc_gather()(rowdist, rowarg, sample_prob)
    return colterm[0, 0] + jnp.sum(partials)
